# baseline (speedup 1.0000x reference)
import functools

import jax
import jax.numpy as jnp
from jax import lax
from jax.experimental import pallas as pl
from jax.experimental.pallas import tpu as pltpu
from jax.experimental.pallas import tpu_sc as plsc

FEAT_NUM = 2
SLICE_NUM = 4
N_HEADS = 8
N_POINTS = 4
D_MODEL = 256
D_HEAD = D_MODEL // N_HEADS
SPATIAL = [(100, 100), (50, 50), (25, 25), (13, 13)]
LSI = [0, 10000, 12500, 13125]
LEN_IN = 13294
BATCH = 2

NFB = FEAT_NUM * BATCH
NW = NFB * N_HEADS
QC = 512
NQP = 13312
NA_CHUNKS = NQP // QC
NPROJ = 5 * 128
CSC = 16
GRP = 4
GQ = GRP * CSC
NG = NQP // GQ
BQ = 512


def _lane_const(vals, dtype):
    lvl = (lax.broadcasted_iota(jnp.int32, (1, 128), 1) % 16) // N_POINTS
    out = jnp.full((1, 128), vals[-1], dtype)
    for l in range(len(vals) - 2, -1, -1):
        out = jnp.where(lvl == l, jnp.asarray(vals[l], dtype), out)
    return out


def _a_body(src_ref, pos_ref, ref_ref, wcat_ref, bcat_ref,
            vt_ref, idx_ref, wgt_ref):
    fb = pl.program_id(0)
    src = src_ref[0, 0] + pos_ref[0, 0]
    proj = lax.dot_general(src.astype(jnp.bfloat16), wcat_ref[0],
                           (((1,), (0,)), ((), ())),
                           preferred_element_type=jnp.float32)
    proj = proj + bcat_ref[0, 0:1, :]
    vt_ref[0] = proj[:, 0:256].astype(jnp.bfloat16)
    sox = proj[:, 256:384]
    soy = proj[:, 384:512]
    e = jnp.exp(proj[:, 512:640])
    jj = lax.broadcasted_iota(jnp.int32, (128, 8), 0)
    hh = lax.broadcasted_iota(jnp.int32, (128, 8), 1)
    msk = (jj // 16 == hh).astype(jnp.float32)
    denom = lax.dot_general(e, msk, (((1,), (0,)), ((), ())),
                            preferred_element_type=jnp.float32)
    dlane = lax.dot_general(denom, msk, (((1,), (1,)), ((), ())),
                            preferred_element_type=jnp.float32)
    aw = e / dlane
    rxy = ref_ref[0, 0]
    rx = jnp.concatenate([rxy[:, 0:16]] * N_HEADS, axis=-1)
    ry = jnp.concatenate([rxy[:, 16:32]] * N_HEADS, axis=-1)
    wvals = [w for (h, w) in SPATIAL]
    hvals = [h for (h, w) in SPATIAL]
    wf = _lane_const(wvals, jnp.float32)
    hf = _lane_const(hvals, jnp.float32)
    wi = _lane_const(wvals, jnp.int32)
    hi = _lane_const(hvals, jnp.int32)
    li = _lane_const(LSI, jnp.int32)
    hl = lax.broadcasted_iota(jnp.int32, (1, 128), 1) // 16
    x = rx * wf + sox - 0.5
    y = ry * hf + soy - 0.5
    x0f = jnp.floor(x)
    y0f = jnp.floor(y)
    fx = x - x0f
    fy = y - y0f
    x0 = x0f.astype(jnp.int32)
    y0 = y0f.astype(jnp.int32)
    x1 = x0 + 1
    y1 = y0 + 1
    vx0 = (x0 >= 0) & (x0 < wi)
    vx1 = (x1 >= 0) & (x1 < wi)
    vy0 = (y0 >= 0) & (y0 < hi)
    vy1 = (y1 >= 0) & (y1 < hi)
    cy0 = jnp.clip(y0, 0, hi - 1)
    cy1 = jnp.clip(y1, 0, hi - 1)
    px = jnp.clip(x0, 0, wi - 2)
    base = li + fb * NQP
    r0 = base + cy0 * wi
    r1 = base + cy1 * wi
    idx_ref[0, 0] = (r0 + px) * 8 + hl
    idx_ref[0, 1] = (r1 + px) * 8 + hl
    gx0 = 1.0 - fx
    gy0 = 1.0 - fy
    zero = jnp.zeros_like(aw)
    w00 = aw * gx0 * gy0 * (vx0 & vy0).astype(jnp.float32)
    w01 = aw * fx * gy0 * (vx1 & vy0).astype(jnp.float32)
    w10 = aw * gx0 * fy * (vx0 & vy1).astype(jnp.float32)
    w11 = aw * fx * fy * (vx1 & vy1).astype(jnp.float32)
    x0_at_l = x0 == px
    x1_at_l = x1 == px
    wgt_ref[0, 0] = (jnp.where(x0_at_l, w00, zero)
                     + jnp.where(x1_at_l, w01, zero))
    wgt_ref[0, 1] = (jnp.where(x0_at_l, zero, w00)
                     + jnp.where(x1_at_l, zero, w01))
    wgt_ref[0, 2] = (jnp.where(x0_at_l, w10, zero)
                     + jnp.where(x1_at_l, w11, zero))
    wgt_ref[0, 3] = (jnp.where(x0_at_l, zero, w10)
                     + jnp.where(x1_at_l, zero, w11))


def _phase_a(src_pad, pos_pad, refxy, wcat, bcat, interpret=False):
    grid = (NFB, NA_CHUNKS)
    return pl.pallas_call(
        _a_body,
        grid=grid,
        in_specs=[
            pl.BlockSpec((1, 1, QC, D_MODEL),
                         lambda fb, qi: (fb // 2, fb % 2, qi, 0)),
            pl.BlockSpec((1, 1, QC, D_MODEL),
                         lambda fb, qi: (fb // 2, fb % 2, qi, 0)),
            pl.BlockSpec((1, 1, QC, 32),
                         lambda fb, qi: (fb // 2, fb % 2, qi, 0)),
            pl.BlockSpec((1, D_MODEL, NPROJ), lambda fb, qi: (fb // 2, 0, 0)),
            pl.BlockSpec((1, 8, NPROJ), lambda fb, qi: (fb // 2, 0, 0)),
        ],
        out_specs=[
            pl.BlockSpec((1, QC, D_MODEL), lambda fb, qi: (fb, qi, 0)),
            pl.BlockSpec((1, 2, QC, 128), lambda fb, qi: (fb, 0, qi, 0)),
            pl.BlockSpec((1, 4, QC, 128), lambda fb, qi: (fb, 0, qi, 0)),
        ],
        out_shape=[
            jax.ShapeDtypeStruct((NFB, NQP, D_MODEL), jnp.bfloat16),
            jax.ShapeDtypeStruct((NFB, 2, NQP, 128), jnp.int32),
            jax.ShapeDtypeStruct((NFB, 4, NQP, 128), jnp.float32),
        ],
        interpret=interpret,
    )(src_pad, pos_pad, refxy, wcat, bcat)


def _sc_gather_mac(vt_flat, idx4, wgt4):
    mesh = plsc.VectorSubcoreMesh(core_axis_name="c", subcore_axis_name="s")

    @functools.partial(
        pl.kernel,
        mesh=mesh,
        compiler_params=pltpu.CompilerParams(use_tc_tiling_on_sc=False,
                                             needs_layout_passes=False),
        out_type=jax.ShapeDtypeStruct((NFB, NQP, D_MODEL), jnp.float32),
        scratch_types=[
            pltpu.VMEM((GQ, 32), jnp.int32),
            pltpu.VMEM((GQ, 32), jnp.int32),
            pltpu.VMEM((GQ, 64), jnp.float32),
            pltpu.VMEM((GQ, 64), jnp.float32),
            pltpu.VMEM((CSC * 32, 64), jnp.bfloat16),
            pltpu.VMEM((CSC * 32, 64), jnp.bfloat16),
            pltpu.VMEM((GQ, 32), jnp.float32),
            pltpu.SemaphoreType.DMA,
            pltpu.SemaphoreType.DMA,
            pltpu.SemaphoreType.DMA,
            pltpu.SemaphoreType.DMA,
        ],
    )
    def k(vt_hbm, idx_hbm, wgt_hbm, out_hbm,
          ig0, ig1, wg0, wg1, r0, r1, outg, sg0, sg1, sr0, sr1):
        igs, wgs, rs = [ig0, ig1], [wg0, wg1], [r0, r1]
        sgs, srs = [sg0, sg1], [sr0, sr1]
        nc = plsc.get_sparse_core_info().num_cores
        wid = lax.axis_index("s") * nc + lax.axis_index("c")
        fb = wid // N_HEADS
        hd = wid % N_HEADS

        def group_copies(g, b, make_only):
            q0 = g * GQ
            f = pltpu.make_async_copy if make_only else pltpu.async_copy
            cps = []
            for c in range(2):
                cps.append(f(idx_hbm.at[fb, c, pl.ds(q0, GQ),
                                        pl.ds(hd * 16, 16)],
                             igs[b].at[:, pl.ds(c * 16, 16)], sgs[b]))
            for c in range(4):
                cps.append(f(wgt_hbm.at[fb, c, pl.ds(q0, GQ),
                                        pl.ds(hd * 16, 16)],
                             wgs[b].at[:, pl.ds(c * 16, 16)], sgs[b]))
            return cps

        def fire(ig, ch, q):
            return [
                pltpu.async_copy(vt_hbm.at[ig.at[ch * CSC + lq]],
                                 rs[q].at[pl.ds(lq * 32, 32)], srs[q])
                for lq in range(CSC)
            ]

        def do_group(g, p):
            @pl.when(g + 1 < NG)
            def _():
                group_copies(g + 1, 1 - p, False)

            pending = fire(igs[p], 0, 0)
            for ch in range(GRP):
                q = ch % 2
                nxt = fire(igs[p], ch + 1, 1 - q) if ch + 1 < GRP else []
                for cp in pending:
                    cp.wait()
                pending = nxt

                @plsc.parallel_loop(0, CSC, 1, unroll=2)
                def q_body(qq):
                    acc0 = jnp.zeros((16,), jnp.float32)
                    acc1 = jnp.zeros((16,), jnp.float32)
                    for t in range(2):
                        wl = wgs[p][ch * CSC + qq, pl.ds(t * 32, 16)]
                        wr = wgs[p][ch * CSC + qq, pl.ds(t * 32 + 16, 16)]
                        for e2 in range(16):
                            jx = qq * 32 + t * 16 + e2
                            wql = wl[e2]
                            wqr = wr[e2]
                            la, ha = plsc.unpack(
                                rs[q][jx, pl.ds(0, 32)],
                                format=plsc.PackFormat.INTERLEAVED)
                            lb, hb = plsc.unpack(
                                rs[q][jx, pl.ds(32, 32)],
                                format=plsc.PackFormat.INTERLEAVED)
                            acc0 = acc0 + wql * la + wqr * lb
                            acc1 = acc1 + wql * ha + wqr * hb
                    outg[ch * CSC + qq, pl.ds(0, 16)] = acc0
                    outg[ch * CSC + qq, pl.ds(16, 16)] = acc1
            pltpu.sync_copy(outg,
                            out_hbm.at[fb, pl.ds(g * GQ, GQ),
                                       pl.ds(hd * 32, 32)])

        for cp in group_copies(0, 0, False):
            cp.wait()

        def pair_body(go, carry):
            for b in range(2):
                g = go * 2 + b

                @pl.when(g > 0)
                def _():
                    for cp in group_copies(g, b, True):
                        cp.wait()

                do_group(g, b)
            return carry

        lax.fori_loop(0, NG // 2, pair_body, 0)

    return k(vt_flat, idx4, wgt4)


def _finish_body(srcs_ref, pos_ref, attn_ref, ow_ref, ob_ref, lw_ref, lb_ref,
                 o_ref):
    src = srcs_ref[0, 0] + pos_ref[0, 0]
    y = src + lax.dot_general(
        attn_ref[0], ow_ref[0], (((1,), (1,)), ((), ())),
        preferred_element_type=jnp.float32) + ob_ref[0, 0:1, :]
    mu = jnp.mean(y, axis=-1, keepdims=True)
    var = jnp.mean((y - mu) ** 2, axis=-1, keepdims=True)
    o_ref[0] = (y - mu) * lax.rsqrt(var + 1e-5) * lw_ref[...] + lb_ref[...]


def _finish(src_pad, pos_pad, attn, ow_s, ob_s, ln_w, ln_b, interpret=False):
    grid = (NFB, NA_CHUNKS)
    return pl.pallas_call(
        _finish_body,
        grid=grid,
        in_specs=[
            pl.BlockSpec((1, 1, BQ, D_MODEL),
                         lambda fb, qi: (fb // 2, fb % 2, qi, 0)),
            pl.BlockSpec((1, 1, BQ, D_MODEL),
                         lambda fb, qi: (fb // 2, fb % 2, qi, 0)),
            pl.BlockSpec((1, BQ, D_MODEL), lambda fb, qi: (fb, qi, 0)),
            pl.BlockSpec((1, D_MODEL, D_MODEL), lambda fb, qi: (fb // 2, 0, 0)),
            pl.BlockSpec((1, 8, D_MODEL), lambda fb, qi: (fb // 2, 0, 0)),
            pl.BlockSpec((D_MODEL,), lambda fb, qi: (0,)),
            pl.BlockSpec((D_MODEL,), lambda fb, qi: (0,)),
        ],
        out_specs=pl.BlockSpec((1, BQ, D_MODEL), lambda fb, qi: (fb, qi, 0)),
        out_shape=jax.ShapeDtypeStruct((NFB, LEN_IN, D_MODEL), jnp.float32),
        interpret=interpret,
    )(src_pad, pos_pad, attn, ow_s, ob_s, ln_w, ln_b)


def _prep_weights(params):
    wcats, bcats, ows, obs = [], [], [], []
    for f in range(FEAT_NUM):
        p = params[f]
        sow = p["so_w"].reshape(N_HEADS, 16, 2, D_MODEL)
        vw = (p["value_w"].reshape(N_HEADS, 2, 16, D_MODEL)
              .transpose(0, 2, 1, 3).reshape(D_MODEL, D_MODEL))
        vb = (p["value_b"].reshape(N_HEADS, 2, 16)
              .transpose(0, 2, 1).reshape(D_MODEL))
        wc = jnp.concatenate([
            vw,
            sow[:, :, 0, :].reshape(128, D_MODEL),
            sow[:, :, 1, :].reshape(128, D_MODEL),
            p["aw_w"],
        ], axis=0)
        wcats.append(wc.T)
        sob = p["so_b"].reshape(N_HEADS, 16, 2)
        bc = jnp.concatenate([
            vb, sob[:, :, 0].reshape(128),
            sob[:, :, 1].reshape(128), p["aw_b"]], axis=0)
        bcats.append(jnp.broadcast_to(bc[None, :], (8, NPROJ)))
        ows.append(p["out_w"])
        obs.append(jnp.broadcast_to(p["out_b"][None, :], (8, D_MODEL)))
    wcat = jnp.stack(wcats).astype(jnp.bfloat16)
    bcat = jnp.stack(bcats)
    return wcat, bcat, jnp.stack(ows), jnp.stack(obs)


def kernel(srcs, pos, reference_points, spatial_shapes, level_start_index,
           padding_mask, params, ln_w, ln_b):
    del spatial_shapes, level_start_index, padding_mask
    pad_q = NQP - LEN_IN
    src_pad = jnp.pad(srcs, ((0, 0), (0, 0), (0, pad_q), (0, 0)))
    pos_pad = jnp.pad(pos, ((0, 0), (0, 0), (0, pad_q), (0, 0)))
    rx = jnp.repeat(reference_points[..., 0], N_POINTS, axis=-1)
    ry = jnp.repeat(reference_points[..., 1], N_POINTS, axis=-1)
    refxy = jnp.pad(jnp.concatenate([rx, ry], axis=-1),
                    ((0, 0), (0, 0), (0, pad_q), (0, 0)))
    wcat, bcat, ow_s, ob_s = _prep_weights(params)

    vt, idx4, wgt4 = _phase_a(src_pad, pos_pad, refxy, wcat, bcat)
    vts = jnp.roll(vt, -1, axis=1)
    vt2 = jnp.concatenate([vt.reshape(NFB, NQP, 8, 1, 32),
                           vts.reshape(NFB, NQP, 8, 1, 32)], axis=3)
    vt_flat = vt2.reshape(NFB * NQP * 8, 64)

    attn = _sc_gather_mac(vt_flat, idx4, wgt4)

    out = _finish(src_pad, pos_pad, attn, ow_s, ob_s, ln_w, ln_b)
    return out.reshape(FEAT_NUM, BATCH, LEN_IN, D_MODEL)


# speedup vs baseline: 1.1946x; 1.1946x over previous
import functools

import jax
import jax.numpy as jnp
from jax import lax
from jax.experimental import pallas as pl
from jax.experimental.pallas import tpu as pltpu
from jax.experimental.pallas import tpu_sc as plsc

FEAT_NUM = 2
SLICE_NUM = 4
N_HEADS = 8
N_POINTS = 4
D_MODEL = 256
D_HEAD = D_MODEL // N_HEADS
SPATIAL = [(100, 100), (50, 50), (25, 25), (13, 13)]
LSI = [0, 10000, 12500, 13125]
LEN_IN = 13294
BATCH = 2

NFB = FEAT_NUM * BATCH
NW = NFB * N_HEADS
QC = 512
NQP = 13312
NA_CHUNKS = NQP // QC
NPROJ = 5 * 128
CSC = 16
GRP = 4
GQ = GRP * CSC
NG = NQP // GQ
BQ = 512


def _lane_const(vals, dtype):
    lvl = (lax.broadcasted_iota(jnp.int32, (1, 128), 1) % 16) // N_POINTS
    out = jnp.full((1, 128), vals[-1], dtype)
    for l in range(len(vals) - 2, -1, -1):
        out = jnp.where(lvl == l, jnp.asarray(vals[l], dtype), out)
    return out


def _a_body(src_ref, pos_ref, ref_ref, wcat_ref, bcat_ref,
            vt_ref, idx_ref, wgt_ref):
    fb = pl.program_id(0)
    src = src_ref[0, 0] + pos_ref[0, 0]
    proj = lax.dot_general(src.astype(jnp.bfloat16), wcat_ref[0],
                           (((1,), (0,)), ((), ())),
                           preferred_element_type=jnp.float32)
    proj = proj + bcat_ref[0, 0:1, :]
    vt_ref[0] = proj[:, 0:256].astype(jnp.bfloat16)
    sox = proj[:, 256:384]
    soy = proj[:, 384:512]
    e = jnp.exp(proj[:, 512:640])
    jj = lax.broadcasted_iota(jnp.int32, (128, 8), 0)
    hh = lax.broadcasted_iota(jnp.int32, (128, 8), 1)
    msk = (jj // 16 == hh).astype(jnp.float32)
    denom = lax.dot_general(e, msk, (((1,), (0,)), ((), ())),
                            preferred_element_type=jnp.float32)
    dlane = lax.dot_general(denom, msk, (((1,), (1,)), ((), ())),
                            preferred_element_type=jnp.float32)
    aw = e / dlane
    rxy = ref_ref[0, 0]
    rx = jnp.concatenate([rxy[:, 0:16]] * N_HEADS, axis=-1)
    ry = jnp.concatenate([rxy[:, 16:32]] * N_HEADS, axis=-1)
    wvals = [w for (h, w) in SPATIAL]
    hvals = [h for (h, w) in SPATIAL]
    wf = _lane_const(wvals, jnp.float32)
    hf = _lane_const(hvals, jnp.float32)
    wi = _lane_const(wvals, jnp.int32)
    hi = _lane_const(hvals, jnp.int32)
    li = _lane_const(LSI, jnp.int32)
    hl = lax.broadcasted_iota(jnp.int32, (1, 128), 1) // 16
    x = rx * wf + sox - 0.5
    y = ry * hf + soy - 0.5
    x0f = jnp.floor(x)
    y0f = jnp.floor(y)
    fx = x - x0f
    fy = y - y0f
    x0 = x0f.astype(jnp.int32)
    y0 = y0f.astype(jnp.int32)
    x1 = x0 + 1
    y1 = y0 + 1
    vx0 = (x0 >= 0) & (x0 < wi)
    vx1 = (x1 >= 0) & (x1 < wi)
    vy0 = (y0 >= 0) & (y0 < hi)
    vy1 = (y1 >= 0) & (y1 < hi)
    cx0 = jnp.clip(x0, 0, wi - 1)
    cx1 = jnp.clip(x1, 0, wi - 1)
    cy0 = jnp.clip(y0, 0, hi - 1)
    cy1 = jnp.clip(y1, 0, hi - 1)
    base = li + fb * NQP
    r0 = base + cy0 * wi
    r1 = base + cy1 * wi
    idx_ref[0, 0] = (r0 + cx0) * 8 + hl
    idx_ref[0, 1] = (r0 + cx1) * 8 + hl
    idx_ref[0, 2] = (r1 + cx0) * 8 + hl
    idx_ref[0, 3] = (r1 + cx1) * 8 + hl
    gx0 = 1.0 - fx
    gy0 = 1.0 - fy
    wgt_ref[0, 0] = aw * gx0 * gy0 * (vx0 & vy0).astype(jnp.float32)
    wgt_ref[0, 1] = aw * fx * gy0 * (vx1 & vy0).astype(jnp.float32)
    wgt_ref[0, 2] = aw * gx0 * fy * (vx0 & vy1).astype(jnp.float32)
    wgt_ref[0, 3] = aw * fx * fy * (vx1 & vy1).astype(jnp.float32)


def _phase_a(src_pad, pos_pad, refxy, wcat, bcat, interpret=False):
    grid = (NFB, NA_CHUNKS)
    return pl.pallas_call(
        _a_body,
        grid=grid,
        in_specs=[
            pl.BlockSpec((1, 1, QC, D_MODEL),
                         lambda fb, qi: (fb // 2, fb % 2, qi, 0)),
            pl.BlockSpec((1, 1, QC, D_MODEL),
                         lambda fb, qi: (fb // 2, fb % 2, qi, 0)),
            pl.BlockSpec((1, 1, QC, 32),
                         lambda fb, qi: (fb // 2, fb % 2, qi, 0)),
            pl.BlockSpec((1, D_MODEL, NPROJ), lambda fb, qi: (fb // 2, 0, 0)),
            pl.BlockSpec((1, 8, NPROJ), lambda fb, qi: (fb // 2, 0, 0)),
        ],
        out_specs=[
            pl.BlockSpec((1, QC, D_MODEL), lambda fb, qi: (fb, qi, 0)),
            pl.BlockSpec((1, 4, QC, 128), lambda fb, qi: (fb, 0, qi, 0)),
            pl.BlockSpec((1, 4, QC, 128), lambda fb, qi: (fb, 0, qi, 0)),
        ],
        out_shape=[
            jax.ShapeDtypeStruct((NFB, NQP, D_MODEL), jnp.bfloat16),
            jax.ShapeDtypeStruct((NFB, 4, NQP, 128), jnp.int32),
            jax.ShapeDtypeStruct((NFB, 4, NQP, 128), jnp.float32),
        ],
        interpret=interpret,
    )(src_pad, pos_pad, refxy, wcat, bcat)


def _sc_gather_mac(vt_flat, idx4, wgt4):
    mesh = plsc.VectorSubcoreMesh(core_axis_name="c", subcore_axis_name="s")

    @functools.partial(
        pl.kernel,
        mesh=mesh,
        compiler_params=pltpu.CompilerParams(use_tc_tiling_on_sc=False,
                                             needs_layout_passes=False),
        out_type=jax.ShapeDtypeStruct((NFB, NQP, D_MODEL), jnp.float32),
        scratch_types=[
            pltpu.VMEM((GQ, 64), jnp.int32),
            pltpu.VMEM((GQ, 64), jnp.int32),
            pltpu.VMEM((GQ, 64), jnp.float32),
            pltpu.VMEM((GQ, 64), jnp.float32),
            pltpu.VMEM((CSC * 64, 32), jnp.bfloat16),
            pltpu.VMEM((CSC * 64, 32), jnp.bfloat16),
            pltpu.VMEM((GQ, 32), jnp.float32),
            pltpu.SemaphoreType.DMA,
            pltpu.SemaphoreType.DMA,
            pltpu.SemaphoreType.DMA,
            pltpu.SemaphoreType.DMA,
        ],
    )
    def k(vt_hbm, idx_hbm, wgt_hbm, out_hbm,
          ig0, ig1, wg0, wg1, r0, r1, outg, sg0, sg1, sr0, sr1):
        igs, wgs, rs = [ig0, ig1], [wg0, wg1], [r0, r1]
        sgs, srs = [sg0, sg1], [sr0, sr1]
        nc = plsc.get_sparse_core_info().num_cores
        wid = lax.axis_index("s") * nc + lax.axis_index("c")
        fb = wid // N_HEADS
        hd = wid % N_HEADS

        def group_copies(g, b, make_only):
            q0 = g * GQ
            f = pltpu.make_async_copy if make_only else pltpu.async_copy
            cps = []
            for c in range(4):
                cps.append(f(idx_hbm.at[fb, c, pl.ds(q0, GQ),
                                        pl.ds(hd * 16, 16)],
                             igs[b].at[:, pl.ds(c * 16, 16)], sgs[b]))
                cps.append(f(wgt_hbm.at[fb, c, pl.ds(q0, GQ),
                                        pl.ds(hd * 16, 16)],
                             wgs[b].at[:, pl.ds(c * 16, 16)], sgs[b]))
            return cps

        def fire(ig, ch, q):
            return [
                pltpu.async_copy(vt_hbm.at[ig.at[ch * CSC + lq]],
                                 rs[q].at[pl.ds(lq * 64, 64)], srs[q])
                for lq in range(CSC)
            ]

        def do_group(g, p):
            @pl.when(g + 1 < NG)
            def _():
                group_copies(g + 1, 1 - p, False)

            pending = fire(igs[p], 0, 0)
            for ch in range(GRP):
                q = ch % 2
                nxt = fire(igs[p], ch + 1, 1 - q) if ch + 1 < GRP else []
                for cp in pending:
                    cp.wait()
                pending = nxt

                @plsc.parallel_loop(0, CSC, 1, unroll=4)
                def q_body(qq):
                    acc0 = jnp.zeros((16,), jnp.float32)
                    acc1 = jnp.zeros((16,), jnp.float32)
                    for t in range(4):
                        wv = wgs[p][ch * CSC + qq, pl.ds(t * 16, 16)]
                        for e2 in range(16):
                            jx = qq * 64 + t * 16 + e2
                            wq = wv[e2]
                            lo, hi = plsc.unpack(
                                rs[q][jx, :],
                                format=plsc.PackFormat.INTERLEAVED)
                            acc0 = acc0 + wq * lo
                            acc1 = acc1 + wq * hi
                    outg[ch * CSC + qq, pl.ds(0, 16)] = acc0
                    outg[ch * CSC + qq, pl.ds(16, 16)] = acc1
            pltpu.sync_copy(outg,
                            out_hbm.at[fb, pl.ds(g * GQ, GQ),
                                       pl.ds(hd * 32, 32)])

        for cp in group_copies(0, 0, False):
            cp.wait()

        def pair_body(go, carry):
            for b in range(2):
                g = go * 2 + b

                @pl.when(g > 0)
                def _():
                    for cp in group_copies(g, b, True):
                        cp.wait()

                do_group(g, b)
            return carry

        lax.fori_loop(0, NG // 2, pair_body, 0)

    return k(vt_flat, idx4, wgt4)


def _finish_body(srcs_ref, pos_ref, attn_ref, ow_ref, ob_ref, lw_ref, lb_ref,
                 o_ref):
    src = srcs_ref[0, 0] + pos_ref[0, 0]
    y = src + lax.dot_general(
        attn_ref[0], ow_ref[0], (((1,), (1,)), ((), ())),
        preferred_element_type=jnp.float32) + ob_ref[0, 0:1, :]
    mu = jnp.mean(y, axis=-1, keepdims=True)
    var = jnp.mean((y - mu) ** 2, axis=-1, keepdims=True)
    o_ref[0] = (y - mu) * lax.rsqrt(var + 1e-5) * lw_ref[...] + lb_ref[...]


def _finish(src_pad, pos_pad, attn, ow_s, ob_s, ln_w, ln_b, interpret=False):
    grid = (NFB, NA_CHUNKS)
    return pl.pallas_call(
        _finish_body,
        grid=grid,
        in_specs=[
            pl.BlockSpec((1, 1, BQ, D_MODEL),
                         lambda fb, qi: (fb // 2, fb % 2, qi, 0)),
            pl.BlockSpec((1, 1, BQ, D_MODEL),
                         lambda fb, qi: (fb // 2, fb % 2, qi, 0)),
            pl.BlockSpec((1, BQ, D_MODEL), lambda fb, qi: (fb, qi, 0)),
            pl.BlockSpec((1, D_MODEL, D_MODEL), lambda fb, qi: (fb // 2, 0, 0)),
            pl.BlockSpec((1, 8, D_MODEL), lambda fb, qi: (fb // 2, 0, 0)),
            pl.BlockSpec((D_MODEL,), lambda fb, qi: (0,)),
            pl.BlockSpec((D_MODEL,), lambda fb, qi: (0,)),
        ],
        out_specs=pl.BlockSpec((1, BQ, D_MODEL), lambda fb, qi: (fb, qi, 0)),
        out_shape=jax.ShapeDtypeStruct((NFB, LEN_IN, D_MODEL), jnp.float32),
        interpret=interpret,
    )(src_pad, pos_pad, attn, ow_s, ob_s, ln_w, ln_b)


def _prep_weights(params):
    wcats, bcats, ows, obs = [], [], [], []
    for f in range(FEAT_NUM):
        p = params[f]
        sow = p["so_w"].reshape(N_HEADS, 16, 2, D_MODEL)
        vw = (p["value_w"].reshape(N_HEADS, 2, 16, D_MODEL)
              .transpose(0, 2, 1, 3).reshape(D_MODEL, D_MODEL))
        vb = (p["value_b"].reshape(N_HEADS, 2, 16)
              .transpose(0, 2, 1).reshape(D_MODEL))
        wc = jnp.concatenate([
            vw,
            sow[:, :, 0, :].reshape(128, D_MODEL),
            sow[:, :, 1, :].reshape(128, D_MODEL),
            p["aw_w"],
        ], axis=0)
        wcats.append(wc.T)
        sob = p["so_b"].reshape(N_HEADS, 16, 2)
        bc = jnp.concatenate([
            vb, sob[:, :, 0].reshape(128),
            sob[:, :, 1].reshape(128), p["aw_b"]], axis=0)
        bcats.append(jnp.broadcast_to(bc[None, :], (8, NPROJ)))
        ows.append(p["out_w"])
        obs.append(jnp.broadcast_to(p["out_b"][None, :], (8, D_MODEL)))
    wcat = jnp.stack(wcats).astype(jnp.bfloat16)
    bcat = jnp.stack(bcats)
    return wcat, bcat, jnp.stack(ows), jnp.stack(obs)


def kernel(srcs, pos, reference_points, spatial_shapes, level_start_index,
           padding_mask, params, ln_w, ln_b):
    del spatial_shapes, level_start_index, padding_mask
    pad_q = NQP - LEN_IN
    src_pad = jnp.pad(srcs, ((0, 0), (0, 0), (0, pad_q), (0, 0)))
    pos_pad = jnp.pad(pos, ((0, 0), (0, 0), (0, pad_q), (0, 0)))
    rx = jnp.repeat(reference_points[..., 0], N_POINTS, axis=-1)
    ry = jnp.repeat(reference_points[..., 1], N_POINTS, axis=-1)
    refxy = jnp.pad(jnp.concatenate([rx, ry], axis=-1),
                    ((0, 0), (0, 0), (0, pad_q), (0, 0)))
    wcat, bcat, ow_s, ob_s = _prep_weights(params)

    vt, idx4, wgt4 = _phase_a(src_pad, pos_pad, refxy, wcat, bcat)
    vt_flat = vt.reshape(NFB * NQP * 8, 32)

    attn = _sc_gather_mac(vt_flat, idx4, wgt4)

    out = _finish(src_pad, pos_pad, attn, ow_s, ob_s, ln_w, ln_b)
    return out.reshape(FEAT_NUM, BATCH, LEN_IN, D_MODEL)


# speedup vs baseline: 1.2841x; 1.0750x over previous
import functools

import jax
import jax.numpy as jnp
from jax import lax
from jax.experimental import pallas as pl
from jax.experimental.pallas import tpu as pltpu
from jax.experimental.pallas import tpu_sc as plsc

FEAT_NUM = 2
SLICE_NUM = 4
N_HEADS = 8
N_POINTS = 4
D_MODEL = 256
D_HEAD = D_MODEL // N_HEADS
SPATIAL = [(100, 100), (50, 50), (25, 25), (13, 13)]
LSI = [0, 10000, 12500, 13125]
LEN_IN = 13294
BATCH = 2

NFB = FEAT_NUM * BATCH
NW = NFB * N_HEADS
QC = 512
NQP = 13312
NA_CHUNKS = NQP // QC
NPROJ = 5 * 128
CSC = 16
GRP = 4
GQ = GRP * CSC
NG = NQP // GQ
BQ = 512


def _lane_const(vals, dtype):
    lvl = (lax.broadcasted_iota(jnp.int32, (1, 128), 1) % 16) // N_POINTS
    out = jnp.full((1, 128), vals[-1], dtype)
    for l in range(len(vals) - 2, -1, -1):
        out = jnp.where(lvl == l, jnp.asarray(vals[l], dtype), out)
    return out


def _a_body(src_ref, pos_ref, ref_ref, wcat_ref, bcat_ref,
            vt_ref, idx_ref, wgt_ref):
    fb = pl.program_id(0)
    src = src_ref[0, 0] + pos_ref[0, 0]
    proj = lax.dot_general(src.astype(jnp.bfloat16), wcat_ref[0],
                           (((1,), (0,)), ((), ())),
                           preferred_element_type=jnp.float32)
    proj = proj + bcat_ref[0, 0:1, :]
    vt_ref[0] = proj[:, 0:256].astype(jnp.bfloat16)
    sox = proj[:, 256:384]
    soy = proj[:, 384:512]
    e = jnp.exp(proj[:, 512:640])
    jj = lax.broadcasted_iota(jnp.int32, (128, 8), 0)
    hh = lax.broadcasted_iota(jnp.int32, (128, 8), 1)
    msk = (jj // 16 == hh).astype(jnp.float32)
    denom = lax.dot_general(e, msk, (((1,), (0,)), ((), ())),
                            preferred_element_type=jnp.float32)
    dlane = lax.dot_general(denom, msk, (((1,), (1,)), ((), ())),
                            preferred_element_type=jnp.float32)
    aw = e / dlane
    rxy = ref_ref[0, 0]
    rx = jnp.concatenate([rxy[:, 0:16]] * N_HEADS, axis=-1)
    ry = jnp.concatenate([rxy[:, 16:32]] * N_HEADS, axis=-1)
    wvals = [w for (h, w) in SPATIAL]
    hvals = [h for (h, w) in SPATIAL]
    wf = _lane_const(wvals, jnp.float32)
    hf = _lane_const(hvals, jnp.float32)
    wi = _lane_const(wvals, jnp.int32)
    hi = _lane_const(hvals, jnp.int32)
    li = _lane_const(LSI, jnp.int32)
    hl = lax.broadcasted_iota(jnp.int32, (1, 128), 1) // 16
    x = rx * wf + sox - 0.5
    y = ry * hf + soy - 0.5
    x0f = jnp.floor(x)
    y0f = jnp.floor(y)
    fx = x - x0f
    fy = y - y0f
    x0 = x0f.astype(jnp.int32)
    y0 = y0f.astype(jnp.int32)
    x1 = x0 + 1
    y1 = y0 + 1
    vx0 = (x0 >= 0) & (x0 < wi)
    vx1 = (x1 >= 0) & (x1 < wi)
    vy0 = (y0 >= 0) & (y0 < hi)
    vy1 = (y1 >= 0) & (y1 < hi)
    cx0 = jnp.clip(x0, 0, wi - 1)
    cx1 = jnp.clip(x1, 0, wi - 1)
    cy0 = jnp.clip(y0, 0, hi - 1)
    cy1 = jnp.clip(y1, 0, hi - 1)
    base = li + fb * NQP
    r0 = base + cy0 * wi
    r1 = base + cy1 * wi
    idx_ref[0, 0] = (r0 + cx0) * 8 + hl
    idx_ref[0, 1] = (r0 + cx1) * 8 + hl
    idx_ref[0, 2] = (r1 + cx0) * 8 + hl
    idx_ref[0, 3] = (r1 + cx1) * 8 + hl
    gx0 = 1.0 - fx
    gy0 = 1.0 - fy
    wgt_ref[0, 0] = aw * gx0 * gy0 * (vx0 & vy0).astype(jnp.float32)
    wgt_ref[0, 1] = aw * fx * gy0 * (vx1 & vy0).astype(jnp.float32)
    wgt_ref[0, 2] = aw * gx0 * fy * (vx0 & vy1).astype(jnp.float32)
    wgt_ref[0, 3] = aw * fx * fy * (vx1 & vy1).astype(jnp.float32)


def _phase_a(src_pad, pos_pad, refxy, wcat, bcat, interpret=False):
    grid = (NFB, NA_CHUNKS)
    return pl.pallas_call(
        _a_body,
        grid=grid,
        in_specs=[
            pl.BlockSpec((1, 1, QC, D_MODEL),
                         lambda fb, qi: (fb // 2, fb % 2, qi, 0)),
            pl.BlockSpec((1, 1, QC, D_MODEL),
                         lambda fb, qi: (fb // 2, fb % 2, qi, 0)),
            pl.BlockSpec((1, 1, QC, 32),
                         lambda fb, qi: (fb // 2, fb % 2, qi, 0)),
            pl.BlockSpec((1, D_MODEL, NPROJ), lambda fb, qi: (fb // 2, 0, 0)),
            pl.BlockSpec((1, 8, NPROJ), lambda fb, qi: (fb // 2, 0, 0)),
        ],
        out_specs=[
            pl.BlockSpec((1, QC, D_MODEL), lambda fb, qi: (fb, qi, 0)),
            pl.BlockSpec((1, 4, QC, 128), lambda fb, qi: (fb, 0, qi, 0)),
            pl.BlockSpec((1, 4, QC, 128), lambda fb, qi: (fb, 0, qi, 0)),
        ],
        out_shape=[
            jax.ShapeDtypeStruct((NFB, NQP, D_MODEL), jnp.bfloat16),
            jax.ShapeDtypeStruct((NFB, 4, NQP, 128), jnp.int32),
            jax.ShapeDtypeStruct((NFB, 4, NQP, 128), jnp.float32),
        ],
        interpret=interpret,
    )(src_pad, pos_pad, refxy, wcat, bcat)


def _sc_gather_mac(vt_flat, idx4, wgt4):
    mesh = plsc.VectorSubcoreMesh(core_axis_name="c", subcore_axis_name="s")

    @functools.partial(
        pl.kernel,
        mesh=mesh,
        compiler_params=pltpu.CompilerParams(use_tc_tiling_on_sc=False,
                                             needs_layout_passes=False),
        out_type=jax.ShapeDtypeStruct((NFB, NQP, D_MODEL), jnp.float32),
        scratch_types=[
            pltpu.VMEM((GQ, 64), jnp.int32),
            pltpu.VMEM((GQ, 64), jnp.int32),
            pltpu.VMEM((GQ, 64), jnp.float32),
            pltpu.VMEM((GQ, 64), jnp.float32),
            pltpu.VMEM((CSC * 64, 32), jnp.bfloat16),
            pltpu.VMEM((CSC * 64, 32), jnp.bfloat16),
            pltpu.VMEM((GQ, 32), jnp.float32),
            pltpu.SemaphoreType.DMA,
            pltpu.SemaphoreType.DMA,
            pltpu.SemaphoreType.DMA,
            pltpu.SemaphoreType.DMA,
        ],
    )
    def k(vt_hbm, idx_hbm, wgt_hbm, out_hbm,
          ig0, ig1, wg0, wg1, r0, r1, outg, sg0, sg1, sr0, sr1):
        igs, wgs, rs = [ig0, ig1], [wg0, wg1], [r0, r1]
        sgs, srs = [sg0, sg1], [sr0, sr1]
        nc = plsc.get_sparse_core_info().num_cores
        wid = lax.axis_index("s") * nc + lax.axis_index("c")
        fb = wid // N_HEADS
        hd = wid % N_HEADS

        def group_copies(g, b, make_only):
            q0 = g * GQ
            f = pltpu.make_async_copy if make_only else pltpu.async_copy
            cps = []
            for c in range(4):
                cps.append(f(idx_hbm.at[fb, c, pl.ds(q0, GQ),
                                        pl.ds(hd * 16, 16)],
                             igs[b].at[:, pl.ds(c * 16, 16)], sgs[b]))
                cps.append(f(wgt_hbm.at[fb, c, pl.ds(q0, GQ),
                                        pl.ds(hd * 16, 16)],
                             wgs[b].at[:, pl.ds(c * 16, 16)], sgs[b]))
            return cps

        def fire(ig, ch, q):
            return [
                pltpu.async_copy(vt_hbm.at[ig.at[ch * CSC + lq]],
                                 rs[q].at[pl.ds(lq * 64, 64)], srs[q])
                for lq in range(CSC)
            ]

        def do_group(g, p):
            @pl.when(g + 1 < NG)
            def _():
                group_copies(g + 1, 1 - p, False)

            pending = fire(igs[p], 0, 0)
            for ch in range(GRP):
                q = ch % 2
                nxt = fire(igs[p], ch + 1, 1 - q) if ch + 1 < GRP else []
                for cp in pending:
                    cp.wait()
                pending = nxt

                @plsc.parallel_loop(0, CSC, 1, unroll=2)
                def q_body(qq):
                    acc0 = jnp.zeros((16,), jnp.float32)
                    acc1 = jnp.zeros((16,), jnp.float32)
                    for t in range(4):
                        wv = wgs[p][ch * CSC + qq, pl.ds(t * 16, 16)]
                        for e2 in range(16):
                            jx = qq * 64 + t * 16 + e2
                            wq = wv[e2]
                            lo, hi = plsc.unpack(
                                rs[q][jx, :],
                                format=plsc.PackFormat.INTERLEAVED)
                            acc0 = acc0 + wq * lo
                            acc1 = acc1 + wq * hi
                    outg[ch * CSC + qq, pl.ds(0, 16)] = acc0
                    outg[ch * CSC + qq, pl.ds(16, 16)] = acc1
            pltpu.sync_copy(outg,
                            out_hbm.at[fb, pl.ds(g * GQ, GQ),
                                       pl.ds(hd * 32, 32)])

        for cp in group_copies(0, 0, False):
            cp.wait()

        def pair_body(go, carry):
            for b in range(2):
                g = go * 2 + b

                @pl.when(g > 0)
                def _():
                    for cp in group_copies(g, b, True):
                        cp.wait()

                do_group(g, b)
            return carry

        lax.fori_loop(0, NG // 2, pair_body, 0)

    return k(vt_flat, idx4, wgt4)


def _finish_body(srcs_ref, pos_ref, attn_ref, ow_ref, ob_ref, lw_ref, lb_ref,
                 o_ref):
    src = srcs_ref[0, 0] + pos_ref[0, 0]
    y = src + lax.dot_general(
        attn_ref[0], ow_ref[0], (((1,), (1,)), ((), ())),
        preferred_element_type=jnp.float32) + ob_ref[0, 0:1, :]
    mu = jnp.mean(y, axis=-1, keepdims=True)
    var = jnp.mean((y - mu) ** 2, axis=-1, keepdims=True)
    o_ref[0] = (y - mu) * lax.rsqrt(var + 1e-5) * lw_ref[...] + lb_ref[...]


def _finish(src_pad, pos_pad, attn, ow_s, ob_s, ln_w, ln_b, interpret=False):
    grid = (NFB, NA_CHUNKS)
    return pl.pallas_call(
        _finish_body,
        grid=grid,
        in_specs=[
            pl.BlockSpec((1, 1, BQ, D_MODEL),
                         lambda fb, qi: (fb // 2, fb % 2, qi, 0)),
            pl.BlockSpec((1, 1, BQ, D_MODEL),
                         lambda fb, qi: (fb // 2, fb % 2, qi, 0)),
            pl.BlockSpec((1, BQ, D_MODEL), lambda fb, qi: (fb, qi, 0)),
            pl.BlockSpec((1, D_MODEL, D_MODEL), lambda fb, qi: (fb // 2, 0, 0)),
            pl.BlockSpec((1, 8, D_MODEL), lambda fb, qi: (fb // 2, 0, 0)),
            pl.BlockSpec((D_MODEL,), lambda fb, qi: (0,)),
            pl.BlockSpec((D_MODEL,), lambda fb, qi: (0,)),
        ],
        out_specs=pl.BlockSpec((1, BQ, D_MODEL), lambda fb, qi: (fb, qi, 0)),
        out_shape=jax.ShapeDtypeStruct((NFB, LEN_IN, D_MODEL), jnp.float32),
        interpret=interpret,
    )(src_pad, pos_pad, attn, ow_s, ob_s, ln_w, ln_b)


def _prep_weights(params):
    wcats, bcats, ows, obs = [], [], [], []
    for f in range(FEAT_NUM):
        p = params[f]
        sow = p["so_w"].reshape(N_HEADS, 16, 2, D_MODEL)
        vw = (p["value_w"].reshape(N_HEADS, 2, 16, D_MODEL)
              .transpose(0, 2, 1, 3).reshape(D_MODEL, D_MODEL))
        vb = (p["value_b"].reshape(N_HEADS, 2, 16)
              .transpose(0, 2, 1).reshape(D_MODEL))
        wc = jnp.concatenate([
            vw,
            sow[:, :, 0, :].reshape(128, D_MODEL),
            sow[:, :, 1, :].reshape(128, D_MODEL),
            p["aw_w"],
        ], axis=0)
        wcats.append(wc.T)
        sob = p["so_b"].reshape(N_HEADS, 16, 2)
        bc = jnp.concatenate([
            vb, sob[:, :, 0].reshape(128),
            sob[:, :, 1].reshape(128), p["aw_b"]], axis=0)
        bcats.append(jnp.broadcast_to(bc[None, :], (8, NPROJ)))
        ows.append(p["out_w"])
        obs.append(jnp.broadcast_to(p["out_b"][None, :], (8, D_MODEL)))
    wcat = jnp.stack(wcats).astype(jnp.bfloat16)
    bcat = jnp.stack(bcats)
    return wcat, bcat, jnp.stack(ows), jnp.stack(obs)


def kernel(srcs, pos, reference_points, spatial_shapes, level_start_index,
           padding_mask, params, ln_w, ln_b):
    del spatial_shapes, level_start_index, padding_mask
    pad_q = NQP - LEN_IN
    src_pad = jnp.pad(srcs, ((0, 0), (0, 0), (0, pad_q), (0, 0)))
    pos_pad = jnp.pad(pos, ((0, 0), (0, 0), (0, pad_q), (0, 0)))
    rx = jnp.repeat(reference_points[..., 0], N_POINTS, axis=-1)
    ry = jnp.repeat(reference_points[..., 1], N_POINTS, axis=-1)
    refxy = jnp.pad(jnp.concatenate([rx, ry], axis=-1),
                    ((0, 0), (0, 0), (0, pad_q), (0, 0)))
    wcat, bcat, ow_s, ob_s = _prep_weights(params)

    vt, idx4, wgt4 = _phase_a(src_pad, pos_pad, refxy, wcat, bcat)
    vt_flat = vt.reshape(NFB * NQP * 8, 32)

    attn = _sc_gather_mac(vt_flat, idx4, wgt4)

    out = _finish(src_pad, pos_pad, attn, ow_s, ob_s, ln_w, ln_b)
    return out.reshape(FEAT_NUM, BATCH, LEN_IN, D_MODEL)
